# baseline (speedup 1.0000x reference)
import functools

import jax
import jax.numpy as jnp
from jax import lax
from jax.experimental import pallas as pl
from jax.experimental.pallas import tpu as pltpu
from jax.experimental.pallas import tpu_sc as plsc

_NUM_FIELDS = 26
_FIELD_DIM = 38462
_OFFS = [f * _FIELD_DIM for f in range(_NUM_FIELDS)]

_BATCH = 16384
_NW = 32
_BPW = _BATCH // _NW
_CW = 128
_QROWS = _BPW // _CW
_SPLITS = [7, 7, 6, 6]

_mesh = plsc.VectorSubcoreMesh(core_axis_name="c", subcore_axis_name="s")


def _make_part(nf):
    nch = nf * _QROWS

    @functools.partial(
        pl.kernel,
        out_type=jax.ShapeDtypeStruct((_BATCH,), jnp.float32),
        mesh=_mesh,
        scratch_types=[
            pltpu.VMEM((nch, _CW), jnp.int32),
            pltpu.VMEM((nch, _CW), jnp.float32),
            pltpu.VMEM((_BPW,), jnp.float32),
            pltpu.SemaphoreType.DMA,
            pltpu.SemaphoreType.DMA,
            pltpu.SemaphoreType.DMA,
            pltpu.SemaphoreType.DMA,
        ],
    )
    def _part(*refs):
        xt_hbm, zd_hbm = refs[0], refs[1]
        tabs = refs[2:2 + nf]
        out_hbm = refs[2 + nf]
        idx_v, rows_v, ob_v = refs[3 + nf:6 + nf]
        qsems = refs[6 + nf:10 + nf]
        wid = lax.axis_index("s") * 2 + lax.axis_index("c")

        pltpu.sync_copy(xt_hbm.at[wid], idx_v)

        for q in range(_QROWS):
            for f in range(nf):
                c = f * _QROWS + q
                pltpu.async_copy(tabs[f].at[idx_v.at[c]], rows_v.at[c],
                                 qsems[q])

        for q in range(_QROWS):
            for f in range(nf):
                pltpu.make_async_copy(zd_hbm, rows_v.at[f * _QROWS + q],
                                      qsems[q]).wait()

            def _reduce(j, carry, q=q):
                sl = pl.ds(j * 16, 16)
                acc = rows_v[q, sl]
                for f in range(1, nf):
                    acc = acc + rows_v[f * _QROWS + q, sl]
                ob_v[pl.ds(q * _CW + j * 16, 16)] = acc
                return carry

            lax.fori_loop(0, _CW // 16, _reduce, 0)

        pltpu.sync_copy(ob_v, out_hbm.at[pl.ds(wid * _BPW, _BPW)])

    return _part


_part7 = _make_part(7)
_part6 = _make_part(6)


def _combine_body(bias_ref, a_ref, b_ref, c_ref, d_ref, out_ref):
    z = (a_ref[...] + b_ref[...]) + (c_ref[...] + d_ref[...]) + bias_ref[0]
    out_ref[...] = 1.0 / (1.0 + jnp.exp(-z))


_tc_combine = pl.pallas_call(
    _combine_body,
    out_shape=jax.ShapeDtypeStruct((_BATCH,), jnp.float32),
    in_specs=[pl.BlockSpec(memory_space=pltpu.SMEM)]
    + [pl.BlockSpec(memory_space=pltpu.VMEM)] * 4,
    out_specs=pl.BlockSpec(memory_space=pltpu.VMEM),
)


def _xt_part(x, f0, nf):
    return (x[:, f0:f0 + nf]
            .astype(jnp.int32)
            .reshape(_NW, _BPW, nf)
            .transpose(0, 2, 1)
            .reshape(_NW, nf * _QROWS, _CW))


def kernel(x, W, bias):
    zd = jnp.zeros((_CW,), jnp.float32)
    parts = []
    f0 = 0
    for nf in _SPLITS:
        tabs = [W[_OFFS[f]:_OFFS[f] + _FIELD_DIM].reshape(-1)
                for f in range(f0, f0 + nf)]
        part = _part7 if nf == 7 else _part6
        parts.append(part(_xt_part(x, f0, nf), zd, *tabs))
        f0 += nf
    return _tc_combine(bias.astype(jnp.float32), *parts)


# speedup vs baseline: 1.0201x; 1.0201x over previous
import functools

import jax
import jax.numpy as jnp
from jax import lax
from jax.experimental import pallas as pl
from jax.experimental.pallas import tpu as pltpu
from jax.experimental.pallas import tpu_sc as plsc

_NUM_FIELDS = 26
_FIELD_DIM = 38462
_OFFS = [f * _FIELD_DIM for f in range(_NUM_FIELDS)]

_BATCH = 16384
_NW = 32
_BPW = _BATCH // _NW
_CW = 128
_QROWS = _BPW // _CW
_SPLITS = [9, 17]

_mesh = plsc.VectorSubcoreMesh(core_axis_name="c", subcore_axis_name="s")


def _make_part(nf):
    nch = nf * _QROWS

    @functools.partial(
        pl.kernel,
        out_type=jax.ShapeDtypeStruct((_BATCH,), jnp.float32),
        mesh=_mesh,
        scratch_types=[
            pltpu.VMEM((nch, _CW), jnp.int32),
            pltpu.VMEM((nch, _CW), jnp.float32),
            pltpu.VMEM((_BPW,), jnp.float32),
            pltpu.SemaphoreType.DMA,
            pltpu.SemaphoreType.DMA,
            pltpu.SemaphoreType.DMA,
            pltpu.SemaphoreType.DMA,
        ],
    )
    def _part(*refs):
        xt_hbm, zd_hbm = refs[0], refs[1]
        tabs = refs[2:2 + nf]
        out_hbm = refs[2 + nf]
        idx_v, rows_v, ob_v = refs[3 + nf:6 + nf]
        qsems = refs[6 + nf:10 + nf]
        wid = lax.axis_index("s") * 2 + lax.axis_index("c")

        pltpu.sync_copy(xt_hbm.at[wid], idx_v)

        for q in range(_QROWS):
            for f in range(nf):
                c = f * _QROWS + q
                pltpu.async_copy(tabs[f].at[idx_v.at[c]], rows_v.at[c],
                                 qsems[q])

        for q in range(_QROWS):
            for f in range(nf):
                pltpu.make_async_copy(zd_hbm, rows_v.at[f * _QROWS + q],
                                      qsems[q]).wait()

            def _reduce(j, carry, q=q):
                sl = pl.ds(j * 16, 16)
                acc = rows_v[q, sl]
                for f in range(1, nf):
                    acc = acc + rows_v[f * _QROWS + q, sl]
                ob_v[pl.ds(q * _CW + j * 16, 16)] = acc
                return carry

            lax.fori_loop(0, _CW // 16, _reduce, 0)

        pltpu.sync_copy(ob_v, out_hbm.at[pl.ds(wid * _BPW, _BPW)])

    return _part


_part9 = _make_part(9)
_part17 = _make_part(17)


def _combine_body(bias_ref, a_ref, b_ref, out_ref):
    z = a_ref[...] + b_ref[...] + bias_ref[0]
    out_ref[...] = 1.0 / (1.0 + jnp.exp(-z))


_tc_combine = pl.pallas_call(
    _combine_body,
    out_shape=jax.ShapeDtypeStruct((_BATCH,), jnp.float32),
    in_specs=[pl.BlockSpec(memory_space=pltpu.SMEM)]
    + [pl.BlockSpec(memory_space=pltpu.VMEM)] * 2,
    out_specs=pl.BlockSpec(memory_space=pltpu.VMEM),
)


def _xt_part(x, f0, nf):
    return (x[:, f0:f0 + nf]
            .astype(jnp.int32)
            .reshape(_NW, _BPW, nf)
            .transpose(0, 2, 1)
            .reshape(_NW, nf * _QROWS, _CW))


def kernel(x, W, bias):
    zd = jnp.zeros((_CW,), jnp.float32)
    parts = []
    f0 = 0
    for nf in _SPLITS:
        tabs = [W[_OFFS[f]:_OFFS[f] + _FIELD_DIM].reshape(-1)
                for f in range(f0, f0 + nf)]
        part = _part9 if nf == 9 else _part17
        parts.append(part(_xt_part(x, f0, nf), zd, *tabs))
        f0 += nf
    return _tc_combine(bias.astype(jnp.float32), *parts)


# speedup vs baseline: 1.0635x; 1.0426x over previous
import functools

import jax
import jax.numpy as jnp
from jax import lax
from jax.experimental import pallas as pl
from jax.experimental.pallas import tpu as pltpu
from jax.experimental.pallas import tpu_sc as plsc

_NUM_FIELDS = 26
_FIELD_DIM = 38462
_OFFS = [f * _FIELD_DIM for f in range(_NUM_FIELDS)]

_BATCH = 16384
_NW = 32
_BPW = _BATCH // _NW
_CW = 128
_QROWS = _BPW // _CW
_NFH = 13
_NCH = _NFH * _QROWS

_mesh = plsc.VectorSubcoreMesh(core_axis_name="c", subcore_axis_name="s")


@functools.partial(
    pl.kernel,
    out_type=jax.ShapeDtypeStruct((_BATCH,), jnp.float32),
    mesh=_mesh,
    scratch_types=[
        pltpu.VMEM((_NCH, _CW), jnp.int32),
        pltpu.VMEM((_NCH, _CW), jnp.float32),
        pltpu.VMEM((_BPW,), jnp.float32),
        pltpu.SemaphoreType.DMA,
        pltpu.SemaphoreType.DMA,
        pltpu.SemaphoreType.DMA,
        pltpu.SemaphoreType.DMA,
    ],
)
def _sc_half(xt_hbm, zd_hbm,
             t0, t1, t2, t3, t4, t5, t6, t7, t8, t9, t10, t11, t12,
             out_hbm, idx_v, rows_v, ob_v, sem0, sem1, sem2, sem3):
    wid = lax.axis_index("s") * 2 + lax.axis_index("c")
    tabs = [t0, t1, t2, t3, t4, t5, t6, t7, t8, t9, t10, t11, t12]
    qsems = [sem0, sem1, sem2, sem3]

    pltpu.sync_copy(xt_hbm.at[wid], idx_v)

    for q in range(_QROWS):
        for f in range(_NFH):
            c = f * _QROWS + q
            pltpu.async_copy(tabs[f].at[idx_v.at[c]], rows_v.at[c], qsems[q])

    for q in range(_QROWS):
        for f in range(_NFH):
            pltpu.make_async_copy(zd_hbm, rows_v.at[f * _QROWS + q],
                                  qsems[q]).wait()

        def _reduce(j, carry, q=q):
            sl = pl.ds(j * 16, 16)
            acc = rows_v[q, sl]
            for f in range(1, _NFH):
                acc = acc + rows_v[f * _QROWS + q, sl]
            ob_v[pl.ds(q * _CW + j * 16, 16)] = acc
            return carry

        lax.fori_loop(0, _CW // 16, _reduce, 0)

    pltpu.sync_copy(ob_v, out_hbm.at[pl.ds(wid * _BPW, _BPW)])


def _combine_body(bias_ref, a_ref, b_ref, out_ref):
    z = a_ref[...] + b_ref[...] + bias_ref[0]
    out_ref[...] = 1.0 / (1.0 + jnp.exp(-z))


_tc_combine = pl.pallas_call(
    _combine_body,
    out_shape=jax.ShapeDtypeStruct((_BATCH,), jnp.float32),
    in_specs=[
        pl.BlockSpec(memory_space=pltpu.SMEM),
        pl.BlockSpec(memory_space=pltpu.VMEM),
        pl.BlockSpec(memory_space=pltpu.VMEM),
    ],
    out_specs=pl.BlockSpec(memory_space=pltpu.VMEM),
)


def _xt_half(x, f0):
    return (x[:, f0:f0 + _NFH]
            .astype(jnp.int32)
            .reshape(_NW, _BPW, _NFH)
            .transpose(0, 2, 1)
            .reshape(_NW, _NCH, _CW))


def kernel(x, W, bias):
    tabs = [W[_OFFS[f]:_OFFS[f] + _FIELD_DIM].reshape(-1)
            for f in range(_NUM_FIELDS)]
    zd = jnp.zeros((_CW,), jnp.float32)
    p0 = _sc_half(_xt_half(x, 0), zd, *tabs[:_NFH])
    p1 = _sc_half(_xt_half(x, _NFH), zd, *tabs[_NFH:])
    return _tc_combine(bias.astype(jnp.float32), p0, p1)
